# baseline (speedup 1.0000x reference)
import functools

import jax
import jax.numpy as jnp
from jax import lax
from jax.experimental import pallas as pl
from jax.experimental.pallas import tpu as pltpu
from jax.experimental.pallas import tpu_sc as plsc

NC = 2
NS = 16
NW = NC * NS
LANE = 16
K = 128

_Z16 = None


def _zero_rows(ref, nrows, ncols):
  zv = jnp.zeros((LANE,), jnp.float32)

  def row(i, _):
    def col(c, __):
      ref[i, pl.ds(c * LANE, LANE)] = zv
      return 0
    lax.fori_loop(0, ncols // LANE, col, 0)
    return 0

  lax.fori_loop(0, nrows, row, 0)


def _deg_body(n_pad, rpt, n_chunks, dst_hbm, out_hbm, dst_v, ones_v, zbuf_v,
              deg_sh, sem):
  c = lax.axis_index("c")
  s = lax.axis_index("s")
  wid = c * NS + s

  pltpu.sync_copy(dst_hbm.at[pl.ds(wid * n_chunks, n_chunks)], dst_v)

  ov = jnp.ones((LANE,), jnp.float32)
  zv = jnp.zeros((LANE,), jnp.float32)

  def fill(i, _):
    ones_v[pl.ds(i * LANE, LANE)] = ov
    return 0
  lax.fori_loop(0, K // LANE, fill, 0)

  def zfill(i, _):
    zbuf_v[pl.ds(i * LANE, LANE)] = zv
    return 0
  lax.fori_loop(0, rpt // LANE, zfill, 0)

  pltpu.sync_copy(zbuf_v, deg_sh.at[pl.ds(s * rpt, rpt)])
  plsc.subcore_barrier()

  def chunk(j, _):
    pltpu.sync_copy(ones_v, deg_sh.at[dst_v.at[j]], add=True)
    return 0
  lax.fori_loop(0, n_chunks, chunk, 0)

  plsc.subcore_barrier()

  pltpu.sync_copy(deg_sh.at[pl.ds(s * rpt, rpt)],
                  out_hbm.at[pl.ds(c * n_pad + s * rpt, rpt)])


def _agg_body(n_pad, d, rpt, n_chunks, g_hbm, src_hbm, dst_hbm, out_hbm,
              src_v, dst_v, rows_v, acc_sh, isem0, isem1, gsem0, gsem1,
              ssem0, ssem1):
  c = lax.axis_index("c")
  s = lax.axis_index("s")
  wid = c * NS + s
  base = wid * n_chunks
  isem = (isem0, isem1)
  gsem = (gsem0, gsem1)
  ssem = (ssem0, ssem1)

  def load_src(j, b):
    pltpu.async_copy(src_hbm.at[base + j], src_v.at[b], isem[b])

  def wait_src(b):
    pltpu.make_async_copy(src_hbm.at[0], src_v.at[b], isem[b]).wait()

  def gather(b_src, b_rows):
    pltpu.async_copy(g_hbm.at[src_v.at[b_src]], rows_v.at[b_rows],
                     gsem[b_rows])

  def wait_gather(b):
    pltpu.make_async_copy(g_hbm.at[src_v.at[b]], rows_v.at[b],
                          gsem[b]).wait()

  def scatter(j, b):
    pltpu.async_copy(rows_v.at[b], acc_sh.at[dst_v.at[j]], ssem[b],
                     add=True)

  def wait_scatter(b):
    pltpu.make_async_copy(rows_v.at[b], acc_sh.at[pl.ds(0, K)],
                          ssem[b]).wait()

  pltpu.sync_copy(dst_hbm.at[pl.ds(base, n_chunks)], dst_v)

  zv = jnp.zeros((LANE,), jnp.float32)

  def zrow(i, _):
    def zcol(cc, __):
      rows_v[0, i, pl.ds(cc * LANE, LANE)] = zv
      return 0
    lax.fori_loop(0, d // LANE, zcol, 0)
    return 0
  lax.fori_loop(0, K, zrow, 0)
  for j in range(rpt // K):
    pltpu.sync_copy(rows_v.at[0], acc_sh.at[pl.ds(s * rpt + j * K, K)])
  plsc.subcore_barrier()

  load_src(0, 0)
  load_src(1, 1)
  wait_src(0)
  gather(0, 0)
  wait_gather(0)
  scatter(0, 0)
  wait_src(1)
  gather(1, 1)
  load_src(2, 0)

  def pair(i, _):
    for b in (1, 0):
      j = 2 * i + 2 - b
      wait_gather(b)
      scatter(j, b)
      wait_src(1 - b)
      wait_scatter(1 - b)
      gather(1 - b, 1 - b)
      load_src(j + 2, b)
    return 0
  lax.fori_loop(0, (n_chunks - 2) // 2, pair, 0)

  wait_gather(1)
  scatter(n_chunks - 1, 1)
  wait_scatter(0)
  wait_scatter(1)
  wait_src(0)

  plsc.subcore_barrier()

  for j in range(rpt // K):
    pltpu.sync_copy(acc_sh.at[pl.ds(s * rpt + j * K, K)],
                    out_hbm.at[pl.ds(c * n_pad + s * rpt + j * K, K)])


@functools.lru_cache(maxsize=None)
def _make_deg(n_pad, n_chunks):
  rpt = n_pad // NS
  mesh = plsc.VectorSubcoreMesh(core_axis_name="c", subcore_axis_name="s",
                                num_cores=NC, num_subcores=NS)
  return pl.kernel(
      functools.partial(_deg_body, n_pad, rpt, n_chunks),
      out_type=jax.ShapeDtypeStruct((NC * n_pad,), jnp.float32),
      mesh=mesh,
      scratch_types=[
          pltpu.VMEM((n_chunks, K), jnp.int32),
          pltpu.VMEM((K,), jnp.float32),
          pltpu.VMEM((rpt,), jnp.float32),
          pltpu.VMEM_SHARED((n_pad,), jnp.float32),
          pltpu.SemaphoreType.DMA,
      ],
  )


@functools.lru_cache(maxsize=None)
def _make_agg(n_pad, d, n_chunks):
  rpt = n_pad // NS
  mesh = plsc.VectorSubcoreMesh(core_axis_name="c", subcore_axis_name="s",
                                num_cores=NC, num_subcores=NS)
  return pl.kernel(
      functools.partial(_agg_body, n_pad, d, rpt, n_chunks),
      out_type=jax.ShapeDtypeStruct((NC * n_pad, d), jnp.float32),
      mesh=mesh,
      scratch_types=[
          pltpu.VMEM((2, K), jnp.int32),
          pltpu.VMEM((n_chunks, K), jnp.int32),
          pltpu.VMEM((2, K, d), jnp.float32),
          pltpu.VMEM_SHARED((n_pad, d), jnp.float32),
          pltpu.SemaphoreType.DMA,
          pltpu.SemaphoreType.DMA,
          pltpu.SemaphoreType.DMA,
          pltpu.SemaphoreType.DMA,
          pltpu.SemaphoreType.DMA,
          pltpu.SemaphoreType.DMA,
      ],
  )


def _tc1_body(x_ref, w_ref, d0_ref, d1_ref, g_ref):
  dinv = lax.rsqrt(d0_ref[...] + d1_ref[...] + 1.0)
  g_ref[...] = jnp.dot(x_ref[...], w_ref[...],
                       preferred_element_type=jnp.float32) * dinv


def _tc2_body(p0_ref, p1_ref, g_ref, b_ref, w_ref, d0_ref, d1_ref, o_ref):
  dinv = lax.rsqrt(d0_ref[...] + d1_ref[...] + 1.0)
  z = jnp.maximum(
      dinv * (p0_ref[...] + p1_ref[...] + g_ref[...]) + b_ref[...], 0.0)
  o_ref[...] = jnp.dot(z, w_ref[...], preferred_element_type=jnp.float32) * dinv


def _tc3_body(p0_ref, p1_ref, g_ref, b_ref, d0_ref, d1_ref, o_ref):
  dinv = lax.rsqrt(d0_ref[...] + d1_ref[...] + 1.0)
  o_ref[...] = jnp.maximum(
      dinv * (p0_ref[...] + p1_ref[...] + g_ref[...]) + b_ref[...], 0.0)


def _row_spec(bn, d):
  return pl.BlockSpec((bn, d), lambda i: (i, 0))


def _full_spec(shape):
  return pl.BlockSpec(shape, lambda i: tuple(0 for _ in shape))


def kernel(x, edge_index, W1, b1, W2, b2):
  n, d_in = x.shape
  hid = W1.shape[1]
  e = edge_index.shape[1]

  src = edge_index[0].astype(jnp.int32)
  dst = edge_index[1].astype(jnp.int32)

  e_pad = -(-e // (NW * K * 8)) * (NW * K * 8)
  pad = e_pad - e
  if pad:
    src = jnp.concatenate([src, jnp.zeros((pad,), jnp.int32)])
    dst = jnp.concatenate([dst, jnp.full((pad,), n, jnp.int32)])
  n_chunks = e_pad // (NW * K)
  src2d = jnp.concatenate(
      [src.reshape(NW * n_chunks, K), jnp.zeros((8, K), jnp.int32)])
  dst2d = dst.reshape(NW * n_chunks, K)

  n_pad = -(-(n + 1) // (NS * K)) * (NS * K)

  deg = _make_deg(n_pad, n_chunks)(dst2d).reshape(NC, n_pad)
  d0 = deg[0, :n][:, None]
  d1 = deg[1, :n][:, None]

  bn = 2000 if n % 2000 == 0 else n
  grid = (n // bn,)
  b1r = b1.reshape(1, hid)
  b2r = b2.reshape(1, hid)

  g1 = pl.pallas_call(
      _tc1_body,
      grid=grid,
      in_specs=[_row_spec(bn, d_in), _full_spec((d_in, hid)),
                _row_spec(bn, 1), _row_spec(bn, 1)],
      out_specs=_row_spec(bn, hid),
      out_shape=jax.ShapeDtypeStruct((n, hid), jnp.float32),
  )(x, W1, d0, d1)

  agg = _make_agg(n_pad, hid, n_chunks)
  p = agg(g1, src2d, dst2d).reshape(NC, n_pad, hid)

  g2 = pl.pallas_call(
      _tc2_body,
      grid=grid,
      in_specs=[_row_spec(bn, hid), _row_spec(bn, hid), _row_spec(bn, hid),
                _full_spec((1, hid)), _full_spec((hid, hid)),
                _row_spec(bn, 1), _row_spec(bn, 1)],
      out_specs=_row_spec(bn, hid),
      out_shape=jax.ShapeDtypeStruct((n, hid), jnp.float32),
  )(p[0, :n], p[1, :n], g1, b1r, W2, d0, d1)

  q = agg(g2, src2d, dst2d).reshape(NC, n_pad, hid)

  out = pl.pallas_call(
      _tc3_body,
      grid=grid,
      in_specs=[_row_spec(bn, hid), _row_spec(bn, hid), _row_spec(bn, hid),
                _full_spec((1, hid)), _row_spec(bn, 1), _row_spec(bn, 1)],
      out_specs=_row_spec(bn, hid),
      out_shape=jax.ShapeDtypeStruct((n, hid), jnp.float32),
  )(q[0, :n], q[1, :n], g2, b2r, d0, d1)

  return out


# speedup vs baseline: 3.2588x; 3.2588x over previous
import functools

import jax
import jax.numpy as jnp
from jax import lax
from jax.experimental import pallas as pl
from jax.experimental.pallas import tpu as pltpu
from jax.experimental.pallas import tpu_sc as plsc

NC = 2
NS = 16
NW = NC * NS
LANE = 16
K = 128

_Z16 = None


def _zero_rows(ref, nrows, ncols):
  zv = jnp.zeros((LANE,), jnp.float32)

  def row(i, _):
    def col(c, __):
      ref[i, pl.ds(c * LANE, LANE)] = zv
      return 0
    lax.fori_loop(0, ncols // LANE, col, 0)
    return 0

  lax.fori_loop(0, nrows, row, 0)


def _deg_body(n_pad, rpt, n_chunks, dst_hbm, out_hbm, dst_v, ones_v, zbuf_v,
              deg_sh, sem):
  c = lax.axis_index("c")
  s = lax.axis_index("s")
  wid = c * NS + s

  pltpu.sync_copy(dst_hbm.at[pl.ds(wid * n_chunks, n_chunks)], dst_v)

  ov = jnp.ones((LANE,), jnp.float32)
  zv = jnp.zeros((LANE,), jnp.float32)

  def fill(i, _):
    ones_v[pl.ds(i * LANE, LANE)] = ov
    return 0
  lax.fori_loop(0, K // LANE, fill, 0)

  def zfill(i, _):
    zbuf_v[pl.ds(i * LANE, LANE)] = zv
    return 0
  lax.fori_loop(0, rpt // LANE, zfill, 0)

  pltpu.sync_copy(zbuf_v, deg_sh.at[pl.ds(s * rpt, rpt)])
  plsc.subcore_barrier()

  def chunk(j, _):
    pltpu.sync_copy(ones_v, deg_sh.at[dst_v.at[j]], add=True)
    return 0
  lax.fori_loop(0, n_chunks, chunk, 0)

  plsc.subcore_barrier()

  pltpu.sync_copy(deg_sh.at[pl.ds(s * rpt, rpt)],
                  out_hbm.at[pl.ds(c * n_pad + s * rpt, rpt)])


def _agg_body(n_pad, d, rpt, n_chunks, g_hbm, src_hbm, dst_hbm, out_hbm,
              src_v, dst_v, rows_v, acc_sh, isem0, isem1, gsem0, gsem1,
              ssem0, ssem1):
  c = lax.axis_index("c")
  s = lax.axis_index("s")
  wid = c * NS + s
  base = wid * n_chunks
  isem = (isem0, isem1)
  gsem = (gsem0, gsem1)
  ssem = (ssem0, ssem1)

  def load_src(j, b):
    pltpu.async_copy(src_hbm.at[base + j], src_v.at[b], isem[b])

  def wait_src(b):
    pltpu.make_async_copy(src_hbm.at[0], src_v.at[b], isem[b]).wait()

  def gather(b_src, b_rows):
    pltpu.async_copy(g_hbm.at[src_v.at[b_src]], rows_v.at[b_rows],
                     gsem[b_rows])

  def wait_gather(b):
    pltpu.make_async_copy(g_hbm.at[src_v.at[b]], rows_v.at[b],
                          gsem[b]).wait()

  def scatter(j, b):
    pltpu.async_copy(rows_v.at[b], acc_sh.at[dst_v.at[j]], ssem[b],
                     add=True)

  def wait_scatter(b):
    pltpu.make_async_copy(rows_v.at[b], acc_sh.at[pl.ds(0, K)],
                          ssem[b]).wait()

  pltpu.sync_copy(dst_hbm.at[pl.ds(base, n_chunks)], dst_v)

  zv = jnp.zeros((LANE,), jnp.float32)

  def zrow(i, _):
    def zcol(cc, __):
      rows_v[0, i, pl.ds(cc * LANE, LANE)] = zv
      return 0
    lax.fori_loop(0, d // LANE, zcol, 0)
    return 0
  lax.fori_loop(0, K, zrow, 0)
  for j in range(rpt // K):
    pltpu.sync_copy(rows_v.at[0], acc_sh.at[pl.ds(s * rpt + j * K, K)])
  plsc.subcore_barrier()

  load_src(0, 0)
  load_src(1, 1)
  wait_src(0)
  gather(0, 0)
  wait_gather(0)
  scatter(0, 0)
  wait_src(1)
  gather(1, 1)
  load_src(2, 0)

  def pair(i, _):
    for b in (1, 0):
      j = 2 * i + 2 - b
      wait_gather(b)
      scatter(j, b)
      wait_src(1 - b)
      wait_scatter(1 - b)
      gather(1 - b, 1 - b)
      load_src(j + 2, b)
    return 0
  lax.fori_loop(0, (n_chunks - 2) // 2, pair, 0)

  wait_gather(1)
  scatter(n_chunks - 1, 1)
  wait_scatter(0)
  wait_scatter(1)
  wait_src(0)

  plsc.subcore_barrier()

  for j in range(rpt // K):
    pltpu.sync_copy(acc_sh.at[pl.ds(s * rpt + j * K, K)],
                    out_hbm.at[pl.ds(c * n_pad + s * rpt + j * K, K)])


@functools.lru_cache(maxsize=None)
def _make_deg(n_pad, n_chunks):
  rpt = n_pad // NS
  mesh = plsc.VectorSubcoreMesh(core_axis_name="c", subcore_axis_name="s",
                                num_cores=NC, num_subcores=NS)
  return pl.kernel(
      functools.partial(_deg_body, n_pad, rpt, n_chunks),
      out_type=jax.ShapeDtypeStruct((NC * n_pad,), jnp.float32),
      mesh=mesh,
      scratch_types=[
          pltpu.VMEM((n_chunks, K), jnp.int32),
          pltpu.VMEM((K,), jnp.float32),
          pltpu.VMEM((rpt,), jnp.float32),
          pltpu.VMEM_SHARED((n_pad,), jnp.float32),
          pltpu.SemaphoreType.DMA,
      ],
  )


@functools.lru_cache(maxsize=None)
def _make_agg(n_pad, d, n_chunks):
  rpt = n_pad // NS
  mesh = plsc.VectorSubcoreMesh(core_axis_name="c", subcore_axis_name="s",
                                num_cores=NC, num_subcores=NS)
  return pl.kernel(
      functools.partial(_agg_body, n_pad, d, rpt, n_chunks),
      out_type=jax.ShapeDtypeStruct((NC * n_pad, d), jnp.float32),
      mesh=mesh,
      scratch_types=[
          pltpu.VMEM((2, K), jnp.int32),
          pltpu.VMEM((n_chunks, K), jnp.int32),
          pltpu.VMEM((2, K, d), jnp.float32),
          pltpu.VMEM_SHARED((n_pad, d), jnp.float32),
      ] + [pltpu.SemaphoreType.DMA] * 6,
  )


def _tc1_body(x_ref, w_ref, d0_ref, d1_ref, g_ref):
  dinv = lax.rsqrt(d0_ref[...] + d1_ref[...] + 1.0)
  g_ref[...] = jnp.dot(x_ref[...], w_ref[...],
                       preferred_element_type=jnp.float32) * dinv


def _tc2_body(p0_ref, p1_ref, g_ref, b_ref, w_ref, d0_ref, d1_ref, o_ref):
  dinv = lax.rsqrt(d0_ref[...] + d1_ref[...] + 1.0)
  z = jnp.maximum(
      dinv * (p0_ref[...] + p1_ref[...] + g_ref[...]) + b_ref[...], 0.0)
  o_ref[...] = jnp.dot(z, w_ref[...], preferred_element_type=jnp.float32) * dinv


def _tc3_body(p0_ref, p1_ref, g_ref, b_ref, d0_ref, d1_ref, o_ref):
  dinv = lax.rsqrt(d0_ref[...] + d1_ref[...] + 1.0)
  o_ref[...] = jnp.maximum(
      dinv * (p0_ref[...] + p1_ref[...] + g_ref[...]) + b_ref[...], 0.0)


def _row_spec(bn, d):
  return pl.BlockSpec((bn, d), lambda i: (i, 0))


def _full_spec(shape):
  return pl.BlockSpec(shape, lambda i: tuple(0 for _ in shape))


def kernel(x, edge_index, W1, b1, W2, b2):
  n, d_in = x.shape
  hid = W1.shape[1]
  e = edge_index.shape[1]

  src = edge_index[0].astype(jnp.int32)
  dst = edge_index[1].astype(jnp.int32)

  n_pad = -(-(n + 1) // (NS * K)) * (NS * K)

  e_pad = -(-e // (NW * K * 8)) * (NW * K * 8)
  pad = e_pad - e
  if pad:
    spread = jnp.arange(pad, dtype=jnp.int32)
    src = jnp.concatenate([src, spread % n])
    dst = jnp.concatenate([dst, n + spread % (n_pad - n)])
  n_chunks = e_pad // (NW * K)
  src2d = jnp.concatenate(
      [src.reshape(NW * n_chunks, K), jnp.zeros((8, K), jnp.int32)])
  dst2d = dst.reshape(NW * n_chunks, K)

  deg = _make_deg(n_pad, n_chunks)(dst2d).reshape(NC, n_pad)
  d0 = deg[0, :n][:, None]
  d1 = deg[1, :n][:, None]

  bn = 2000 if n % 2000 == 0 else n
  grid = (n // bn,)
  b1r = b1.reshape(1, hid)
  b2r = b2.reshape(1, hid)

  g1 = pl.pallas_call(
      _tc1_body,
      grid=grid,
      in_specs=[_row_spec(bn, d_in), _full_spec((d_in, hid)),
                _row_spec(bn, 1), _row_spec(bn, 1)],
      out_specs=_row_spec(bn, hid),
      out_shape=jax.ShapeDtypeStruct((n, hid), jnp.float32),
  )(x, W1, d0, d1)

  agg = _make_agg(n_pad, hid, n_chunks)
  p = agg(g1, src2d, dst2d).reshape(NC, n_pad, hid)

  g2 = pl.pallas_call(
      _tc2_body,
      grid=grid,
      in_specs=[_row_spec(bn, hid), _row_spec(bn, hid), _row_spec(bn, hid),
                _full_spec((1, hid)), _full_spec((hid, hid)),
                _row_spec(bn, 1), _row_spec(bn, 1)],
      out_specs=_row_spec(bn, hid),
      out_shape=jax.ShapeDtypeStruct((n, hid), jnp.float32),
  )(p[0, :n], p[1, :n], g1, b1r, W2, d0, d1)

  q = agg(g2, src2d, dst2d).reshape(NC, n_pad, hid)

  out = pl.pallas_call(
      _tc3_body,
      grid=grid,
      in_specs=[_row_spec(bn, hid), _row_spec(bn, hid), _row_spec(bn, hid),
                _full_spec((1, hid)), _row_spec(bn, 1), _row_spec(bn, 1)],
      out_specs=_row_spec(bn, hid),
      out_shape=jax.ShapeDtypeStruct((n, hid), jnp.float32),
  )(q[0, :n], q[1, :n], g2, b2r, d0, d1)

  return out


# speedup vs baseline: 3.9676x; 1.2175x over previous
import functools

import jax
import jax.numpy as jnp
from jax import lax
from jax.experimental import pallas as pl
from jax.experimental.pallas import tpu as pltpu
from jax.experimental.pallas import tpu_sc as plsc

NC = 2
NS = 16
NW = NC * NS
LANE = 16
K = 128

_Z16 = None


def _zero_rows(ref, nrows, ncols):
  zv = jnp.zeros((LANE,), jnp.float32)

  def row(i, _):
    def col(c, __):
      ref[i, pl.ds(c * LANE, LANE)] = zv
      return 0
    lax.fori_loop(0, ncols // LANE, col, 0)
    return 0

  lax.fori_loop(0, nrows, row, 0)


def _deg_body(n_pad, rpt, n_chunks, dst_hbm, out_hbm, dst_v, ones_v, zbuf_v,
              deg_sh, sem):
  c = lax.axis_index("c")
  s = lax.axis_index("s")
  wid = c * NS + s

  pltpu.sync_copy(dst_hbm.at[pl.ds(wid * n_chunks, n_chunks)], dst_v)

  ov = jnp.ones((LANE,), jnp.float32)
  zv = jnp.zeros((LANE,), jnp.float32)

  def fill(i, _):
    ones_v[pl.ds(i * LANE, LANE)] = ov
    return 0
  lax.fori_loop(0, K // LANE, fill, 0)

  def zfill(i, _):
    zbuf_v[pl.ds(i * LANE, LANE)] = zv
    return 0
  lax.fori_loop(0, -(-rpt // LANE), zfill, 0)

  pltpu.sync_copy(zbuf_v.at[pl.ds(0, rpt)], deg_sh.at[pl.ds(s * rpt, rpt)])
  plsc.subcore_barrier()

  def chunk(j, _):
    pltpu.sync_copy(ones_v, deg_sh.at[dst_v.at[j]], add=True)
    return 0
  lax.fori_loop(0, n_chunks, chunk, 0)

  plsc.subcore_barrier()

  pltpu.sync_copy(deg_sh.at[pl.ds(s * rpt, rpt)],
                  out_hbm.at[pl.ds(c * n_pad + s * rpt, rpt)])


def _row_chunks(rpt):
  out = []
  off = 0
  while off < rpt:
    out.append((off, min(K, rpt - off)))
    off += K
  return out


def _agg_body(n_pad, d, rpt, n_chunks, g_hbm, src_hbm, dst_hbm, out_hbm,
              src_v, dst_v, rows_v, acc_sh, *sems):
  c = lax.axis_index("c")
  s = lax.axis_index("s")
  base = (c * NS + s) * n_chunks
  isem, dsem, gsem, ssem = sems[0:3], sems[3:6], sems[6:9], sems[9:12]

  def load_src(j, b):
    pltpu.async_copy(src_hbm.at[base + j], src_v.at[b], isem[b])

  def wait_src(b):
    pltpu.make_async_copy(src_hbm.at[0], src_v.at[b], isem[b]).wait()

  def load_dst(j, b):
    pltpu.async_copy(dst_hbm.at[base + j], dst_v.at[b], dsem[b])

  def wait_dst(b):
    pltpu.make_async_copy(dst_hbm.at[0], dst_v.at[b], dsem[b]).wait()

  def gather(j, b):
    pltpu.async_copy(g_hbm.at[src_v.at[b]], rows_v.at[b], gsem[b])

  def wait_gather(b):
    pltpu.make_async_copy(g_hbm.at[src_v.at[b]], rows_v.at[b],
                          gsem[b]).wait()

  def scatter(j, b):
    pltpu.async_copy(rows_v.at[b], acc_sh.at[dst_v.at[b]], ssem[b],
                     add=True)

  def wait_scatter(b):
    pltpu.make_async_copy(rows_v.at[b], acc_sh.at[pl.ds(0, K)],
                          ssem[b]).wait()

  zv = jnp.zeros((LANE,), jnp.float32)

  def zrow(i, _):
    def zcol(cc, __):
      rows_v[0, i, pl.ds(cc * LANE, LANE)] = zv
      return 0
    lax.fori_loop(0, d // LANE, zcol, 0)
    return 0
  lax.fori_loop(0, K, zrow, 0)
  for off, sz in _row_chunks(rpt):
    pltpu.sync_copy(rows_v.at[0, pl.ds(0, sz)],
                    acc_sh.at[pl.ds(s * rpt + off, sz)])
  plsc.subcore_barrier()

  def body(j, b):
    bp2 = (b + 2) % 3
    wait_gather(b)
    wait_dst(b)
    scatter(j, b)
    wait_src(bp2)
    wait_scatter(bp2)
    gather(j + 2, bp2)
    load_src(j + 3, b)
    load_dst(j + 2, bp2)

  nc = n_chunks
  load_src(0, 0)
  load_src(1, 1)
  load_src(2, 2)
  load_dst(0, 0)
  load_dst(1, 1)
  wait_src(0)
  gather(0, 0)
  wait_src(1)
  gather(1, 1)
  wait_gather(0)
  wait_dst(0)
  scatter(0, 0)
  wait_src(2)
  gather(2, 2)
  load_src(3, 0)
  load_dst(2, 2)

  nb = (nc - 3) // 3

  def block(i, _):
    for r in (1, 2, 3):
      body(3 * i + r, r % 3)
    return 0
  lax.fori_loop(0, nb, block, 0)
  for j in range(3 * nb + 1, nc - 2):
    body(j, j % 3)

  b = (nc - 2) % 3
  wait_gather(b)
  wait_dst(b)
  scatter(nc - 2, b)
  wait_scatter((nc - 3) % 3)
  b = (nc - 1) % 3
  wait_gather(b)
  wait_dst(b)
  scatter(nc - 1, b)
  wait_scatter((nc - 2) % 3)
  wait_scatter((nc - 1) % 3)
  wait_src(nc % 3)

  plsc.subcore_barrier()

  for off, sz in _row_chunks(rpt):
    pltpu.sync_copy(acc_sh.at[pl.ds(s * rpt + off, sz)],
                    out_hbm.at[pl.ds(c * n_pad + s * rpt + off, sz)])


@functools.lru_cache(maxsize=None)
def _make_deg(n_pad, n_chunks):
  rpt = n_pad // NS
  mesh = plsc.VectorSubcoreMesh(core_axis_name="c", subcore_axis_name="s",
                                num_cores=NC, num_subcores=NS)
  return pl.kernel(
      functools.partial(_deg_body, n_pad, rpt, n_chunks),
      out_type=jax.ShapeDtypeStruct((NC * n_pad,), jnp.float32),
      mesh=mesh,
      scratch_types=[
          pltpu.VMEM((n_chunks, K), jnp.int32),
          pltpu.VMEM((K,), jnp.float32),
          pltpu.VMEM((-(-rpt // LANE) * LANE,), jnp.float32),
          pltpu.VMEM_SHARED((n_pad,), jnp.float32),
          pltpu.SemaphoreType.DMA,
      ],
  )


@functools.lru_cache(maxsize=None)
def _make_agg(n_pad, d, n_chunks):
  rpt = n_pad // NS
  mesh = plsc.VectorSubcoreMesh(core_axis_name="c", subcore_axis_name="s",
                                num_cores=NC, num_subcores=NS)
  return pl.kernel(
      functools.partial(_agg_body, n_pad, d, rpt, n_chunks),
      out_type=jax.ShapeDtypeStruct((NC * n_pad, d), jnp.float32),
      mesh=mesh,
      scratch_types=[
          pltpu.VMEM((3, K), jnp.int32),
          pltpu.VMEM((3, K), jnp.int32),
          pltpu.VMEM((3, K, d), jnp.float32),
          pltpu.VMEM_SHARED((n_pad, d), jnp.float32),
      ] + [pltpu.SemaphoreType.DMA] * 12,
  )


def _tc1_body(x_ref, w_ref, d0_ref, d1_ref, g_ref):
  dinv = lax.rsqrt(d0_ref[...] + d1_ref[...] + 1.0)
  g_ref[...] = jnp.dot(x_ref[...], w_ref[...],
                       preferred_element_type=jnp.float32) * dinv


def _tc2_body(p0_ref, p1_ref, g_ref, b_ref, w_ref, d0_ref, d1_ref, o_ref):
  dinv = lax.rsqrt(d0_ref[...] + d1_ref[...] + 1.0)
  z = jnp.maximum(
      dinv * (p0_ref[...] + p1_ref[...] + g_ref[...]) + b_ref[...], 0.0)
  o_ref[...] = jnp.dot(z, w_ref[...], preferred_element_type=jnp.float32) * dinv


def _tc3_body(p0_ref, p1_ref, g_ref, b_ref, d0_ref, d1_ref, o_ref):
  dinv = lax.rsqrt(d0_ref[...] + d1_ref[...] + 1.0)
  o_ref[...] = jnp.maximum(
      dinv * (p0_ref[...] + p1_ref[...] + g_ref[...]) + b_ref[...], 0.0)


def _row_spec(bn, d):
  return pl.BlockSpec((bn, d), lambda i: (i, 0))


def _full_spec(shape):
  return pl.BlockSpec(shape, lambda i: tuple(0 for _ in shape))


def kernel(x, edge_index, W1, b1, W2, b2):
  n, d_in = x.shape
  hid = W1.shape[1]
  e = edge_index.shape[1]

  src = edge_index[0].astype(jnp.int32)
  dst = edge_index[1].astype(jnp.int32)

  n_pad = -(-(n + 1) // (NS * 8)) * (NS * 8)
  n_pad_deg = -(-(n + 1) // (NS * 16)) * (NS * 16)

  e_pad = -(-e // (NW * K * 8)) * (NW * K * 8)
  pad = e_pad - e
  if pad:
    spread = jnp.arange(pad, dtype=jnp.int32)
    src = jnp.concatenate([src, spread % n])
    dst = jnp.concatenate([dst, n + spread % (n_pad - n)])
  n_chunks = e_pad // (NW * K)
  src2d = jnp.concatenate(
      [src.reshape(NW * n_chunks, K), jnp.zeros((8, K), jnp.int32)])
  dst2d = dst.reshape(NW * n_chunks, K)

  deg = _make_deg(n_pad_deg, n_chunks)(dst2d).reshape(NC, n_pad_deg)
  d0 = deg[0, :n][:, None]
  d1 = deg[1, :n][:, None]

  bn = 2000 if n % 2000 == 0 else n
  grid = (n // bn,)
  b1r = b1.reshape(1, hid)
  b2r = b2.reshape(1, hid)

  g1 = pl.pallas_call(
      _tc1_body,
      grid=grid,
      in_specs=[_row_spec(bn, d_in), _full_spec((d_in, hid)),
                _row_spec(bn, 1), _row_spec(bn, 1)],
      out_specs=_row_spec(bn, hid),
      out_shape=jax.ShapeDtypeStruct((n, hid), jnp.float32),
  )(x, W1, d0, d1)

  agg = _make_agg(n_pad, hid, n_chunks)
  p = agg(g1, src2d, dst2d).reshape(NC, n_pad, hid)

  g2 = pl.pallas_call(
      _tc2_body,
      grid=grid,
      in_specs=[_row_spec(bn, hid), _row_spec(bn, hid), _row_spec(bn, hid),
                _full_spec((1, hid)), _full_spec((hid, hid)),
                _row_spec(bn, 1), _row_spec(bn, 1)],
      out_specs=_row_spec(bn, hid),
      out_shape=jax.ShapeDtypeStruct((n, hid), jnp.float32),
  )(p[0, :n], p[1, :n], g1, b1r, W2, d0, d1)

  q = agg(g2, src2d, dst2d).reshape(NC, n_pad, hid)

  out = pl.pallas_call(
      _tc3_body,
      grid=grid,
      in_specs=[_row_spec(bn, hid), _row_spec(bn, hid), _row_spec(bn, hid),
                _full_spec((1, hid)), _row_spec(bn, 1), _row_spec(bn, 1)],
      out_specs=_row_spec(bn, hid),
      out_shape=jax.ShapeDtypeStruct((n, hid), jnp.float32),
  )(q[0, :n], q[1, :n], g2, b2r, d0, d1)

  return out


# speedup vs baseline: 4.1771x; 1.0528x over previous
import functools

import jax
import jax.numpy as jnp
from jax import lax
from jax.experimental import pallas as pl
from jax.experimental.pallas import tpu as pltpu
from jax.experimental.pallas import tpu_sc as plsc

NC = 2
NS = 16
NW = NC * NS
LANE = 16
K = 128

_Z16 = None


def _zero_rows(ref, nrows, ncols):
  zv = jnp.zeros((LANE,), jnp.float32)

  def row(i, _):
    def col(c, __):
      ref[i, pl.ds(c * LANE, LANE)] = zv
      return 0
    lax.fori_loop(0, ncols // LANE, col, 0)
    return 0

  lax.fori_loop(0, nrows, row, 0)


def _deg_body(n_pad, rpt, n_chunks, dst_hbm, out_hbm, dst_v, ones_v, zbuf_v,
              deg_sh, sem):
  c = lax.axis_index("c")
  s = lax.axis_index("s")
  wid = c * NS + s

  pltpu.sync_copy(dst_hbm.at[pl.ds(wid * n_chunks, n_chunks)], dst_v)

  ov = jnp.ones((LANE,), jnp.float32)
  zv = jnp.zeros((LANE,), jnp.float32)

  def fill(i, _):
    ones_v[pl.ds(i * LANE, LANE)] = ov
    return 0
  lax.fori_loop(0, K // LANE, fill, 0)

  def zfill(i, _):
    zbuf_v[pl.ds(i * LANE, LANE)] = zv
    return 0
  lax.fori_loop(0, -(-rpt // LANE), zfill, 0)

  pltpu.sync_copy(zbuf_v.at[pl.ds(0, rpt)], deg_sh.at[pl.ds(s * rpt, rpt)])
  plsc.subcore_barrier()

  def chunk(j, _):
    pltpu.sync_copy(ones_v, deg_sh.at[dst_v.at[j]], add=True)
    return 0
  lax.fori_loop(0, n_chunks, chunk, 0)

  plsc.subcore_barrier()

  pltpu.sync_copy(deg_sh.at[pl.ds(s * rpt, rpt)],
                  out_hbm.at[pl.ds(c * n_pad + s * rpt, rpt)])


def _row_chunks(rpt):
  out = []
  off = 0
  while off < rpt:
    out.append((off, min(K, rpt - off)))
    off += K
  return out


def _agg_body(n_pad, d, rpt, n_chunks, g_hbm, src_hbm, dst_hbm, out_hbm,
              src_v, dst_v, rows_v, acc_sh, *sems):
  c = lax.axis_index("c")
  s = lax.axis_index("s")
  base = (c * NS + s) * n_chunks
  isem, dsem, gsem, ssem = sems[0:3], sems[3:6], sems[6:9], sems[9:12]

  def load_src(j, b):
    pltpu.async_copy(src_hbm.at[base + j], src_v.at[b], isem[b])

  def wait_src(b):
    pltpu.make_async_copy(src_hbm.at[0], src_v.at[b], isem[b]).wait()

  def load_dst(j, b):
    pltpu.async_copy(dst_hbm.at[base + j], dst_v.at[b], dsem[b])

  def wait_dst(b):
    pltpu.make_async_copy(dst_hbm.at[0], dst_v.at[b], dsem[b]).wait()

  def gather(j, b):
    pltpu.async_copy(g_hbm.at[src_v.at[b]], rows_v.at[b], gsem[b])

  def wait_gather(b):
    pltpu.make_async_copy(g_hbm.at[src_v.at[b]], rows_v.at[b],
                          gsem[b]).wait()

  def scatter(j, b):
    pltpu.async_copy(rows_v.at[b], acc_sh.at[dst_v.at[b]], ssem[b],
                     add=True)

  def wait_scatter(b):
    pltpu.make_async_copy(rows_v.at[b], acc_sh.at[pl.ds(0, K)],
                          ssem[b]).wait()

  zv = jnp.zeros((LANE,), jnp.float32)

  def zrow(i, _):
    def zcol(cc, __):
      rows_v[0, i, pl.ds(cc * LANE, LANE)] = zv
      return 0
    lax.fori_loop(0, d // LANE, zcol, 0)
    return 0
  lax.fori_loop(0, K, zrow, 0)
  for off, sz in _row_chunks(rpt):
    pltpu.sync_copy(rows_v.at[0, pl.ds(0, sz)],
                    acc_sh.at[pl.ds(s * rpt + off, sz)])
  plsc.subcore_barrier()

  def body(j, b):
    bp2 = (b + 2) % 3
    wait_gather(b)
    wait_dst(b)
    scatter(j, b)
    wait_src(bp2)
    wait_scatter(bp2)
    gather(j + 2, bp2)
    load_src(j + 3, b)
    load_dst(j + 2, bp2)

  nc = n_chunks
  load_src(0, 0)
  load_src(1, 1)
  load_src(2, 2)
  load_dst(0, 0)
  load_dst(1, 1)
  wait_src(0)
  gather(0, 0)
  wait_src(1)
  gather(1, 1)
  wait_gather(0)
  wait_dst(0)
  scatter(0, 0)
  wait_src(2)
  gather(2, 2)
  load_src(3, 0)
  load_dst(2, 2)

  nb = (nc - 3) // 3

  def block(i, _):
    for r in (1, 2, 3):
      body(3 * i + r, r % 3)
    return 0
  lax.fori_loop(0, nb, block, 0)
  for j in range(3 * nb + 1, nc - 2):
    body(j, j % 3)

  b = (nc - 2) % 3
  wait_gather(b)
  wait_dst(b)
  scatter(nc - 2, b)
  wait_scatter((nc - 3) % 3)
  b = (nc - 1) % 3
  wait_gather(b)
  wait_dst(b)
  scatter(nc - 1, b)
  wait_scatter((nc - 2) % 3)
  wait_scatter((nc - 1) % 3)
  wait_src(nc % 3)

  plsc.subcore_barrier()

  for off, sz in _row_chunks(rpt):
    pltpu.sync_copy(acc_sh.at[pl.ds(s * rpt + off, sz)],
                    out_hbm.at[pl.ds(c * n_pad + s * rpt + off, sz)])


@functools.lru_cache(maxsize=None)
def _make_deg(n_pad, n_chunks):
  rpt = n_pad // NS
  mesh = plsc.VectorSubcoreMesh(core_axis_name="c", subcore_axis_name="s",
                                num_cores=NC, num_subcores=NS)
  return pl.kernel(
      functools.partial(_deg_body, n_pad, rpt, n_chunks),
      out_type=jax.ShapeDtypeStruct((NC * n_pad,), jnp.float32),
      mesh=mesh,
      scratch_types=[
          pltpu.VMEM((n_chunks, K), jnp.int32),
          pltpu.VMEM((K,), jnp.float32),
          pltpu.VMEM((-(-rpt // LANE) * LANE,), jnp.float32),
          pltpu.VMEM_SHARED((n_pad,), jnp.float32),
          pltpu.SemaphoreType.DMA,
      ],
  )


@functools.lru_cache(maxsize=None)
def _make_agg(n_pad, d, n_chunks):
  rpt = n_pad // NS
  mesh = plsc.VectorSubcoreMesh(core_axis_name="c", subcore_axis_name="s",
                                num_cores=NC, num_subcores=NS)
  return pl.kernel(
      functools.partial(_agg_body, n_pad, d, rpt, n_chunks),
      out_type=jax.ShapeDtypeStruct((NC * n_pad, d), jnp.float32),
      mesh=mesh,
      scratch_types=[
          pltpu.VMEM((3, K), jnp.int32),
          pltpu.VMEM((3, K), jnp.int32),
          pltpu.VMEM((3, K, d), jnp.float32),
          pltpu.VMEM_SHARED((n_pad, d), jnp.float32),
      ] + [pltpu.SemaphoreType.DMA] * 12,
  )


def _tc1_body(x_ref, w_ref, d0_ref, d1_ref, g_ref):
  dinv = lax.rsqrt(d0_ref[...] + d1_ref[...] + 1.0)
  g_ref[...] = jnp.dot(x_ref[...], w_ref[...],
                       preferred_element_type=jnp.float32) * dinv


def _tc2_body(p0_ref, p1_ref, g_ref, b_ref, w_ref, d0_ref, d1_ref, o_ref):
  dinv = lax.rsqrt(d0_ref[...] + d1_ref[...] + 1.0)
  z = jnp.maximum(
      dinv * (p0_ref[0] + p1_ref[0] + g_ref[...]) + b_ref[...], 0.0)
  o_ref[...] = jnp.dot(z, w_ref[...], preferred_element_type=jnp.float32) * dinv


def _tc3_body(p0_ref, p1_ref, g_ref, b_ref, d0_ref, d1_ref, o_ref):
  dinv = lax.rsqrt(d0_ref[...] + d1_ref[...] + 1.0)
  o_ref[...] = jnp.maximum(
      dinv * (p0_ref[0] + p1_ref[0] + g_ref[...]) + b_ref[...], 0.0)


def _row_spec(bn, d):
  return pl.BlockSpec((bn, d), lambda i: (i, 0))


def _full_spec(shape):
  return pl.BlockSpec(shape, lambda i: tuple(0 for _ in shape))


def kernel(x, edge_index, W1, b1, W2, b2):
  n, d_in = x.shape
  hid = W1.shape[1]
  e = edge_index.shape[1]

  src = edge_index[0].astype(jnp.int32)
  dst = edge_index[1].astype(jnp.int32)

  n_pad = -(-(n + 1) // (NS * 8)) * (NS * 8)
  n_pad_deg = -(-(n + 1) // (NS * 16)) * (NS * 16)

  e_pad = -(-e // (NW * K * 8)) * (NW * K * 8)
  pad = e_pad - e
  if pad:
    spread = jnp.arange(pad, dtype=jnp.int32)
    src = jnp.concatenate([src, spread % n])
    dst = jnp.concatenate([dst, n + spread % (n_pad - n)])
  n_chunks = e_pad // (NW * K)
  src2d = jnp.concatenate(
      [src.reshape(NW * n_chunks, K), jnp.zeros((8, K), jnp.int32)])
  dst2d = dst.reshape(NW * n_chunks, K)

  deg = _make_deg(n_pad_deg, n_chunks)(dst2d).reshape(NC, n_pad_deg)
  d0 = deg[0, :n][:, None]
  d1 = deg[1, :n][:, None]

  bn = 2000 if n % 2000 == 0 else n
  grid = (n // bn,)
  b1r = b1.reshape(1, hid)
  b2r = b2.reshape(1, hid)

  g1 = pl.pallas_call(
      _tc1_body,
      grid=grid,
      in_specs=[_row_spec(bn, d_in), _full_spec((d_in, hid)),
                _row_spec(bn, 1), _row_spec(bn, 1)],
      out_specs=_row_spec(bn, hid),
      out_shape=jax.ShapeDtypeStruct((n, hid), jnp.float32),
  )(x, W1, d0, d1)

  part0 = pl.BlockSpec((1, bn, hid), lambda i: (0, i, 0))
  part1 = pl.BlockSpec((1, bn, hid), lambda i: (1, i, 0))

  agg = _make_agg(n_pad, hid, n_chunks)
  p = agg(g1, src2d, dst2d).reshape(NC, n_pad, hid)

  g2 = pl.pallas_call(
      _tc2_body,
      grid=grid,
      in_specs=[part0, part1, _row_spec(bn, hid),
                _full_spec((1, hid)), _full_spec((hid, hid)),
                _row_spec(bn, 1), _row_spec(bn, 1)],
      out_specs=_row_spec(bn, hid),
      out_shape=jax.ShapeDtypeStruct((n, hid), jnp.float32),
  )(p, p, g1, b1r, W2, d0, d1)

  q = agg(g2, src2d, dst2d).reshape(NC, n_pad, hid)

  out = pl.pallas_call(
      _tc3_body,
      grid=grid,
      in_specs=[part0, part1, _row_spec(bn, hid),
                _full_spec((1, hid)), _row_spec(bn, 1), _row_spec(bn, 1)],
      out_specs=_row_spec(bn, hid),
      out_shape=jax.ShapeDtypeStruct((n, hid), jnp.float32),
  )(q, q, g2, b2r, d0, d1)

  return out


# speedup vs baseline: 4.1958x; 1.0045x over previous
import functools

import jax
import jax.numpy as jnp
from jax import lax
from jax.experimental import pallas as pl
from jax.experimental.pallas import tpu as pltpu
from jax.experimental.pallas import tpu_sc as plsc

NC = 2
NS = 16
NW = NC * NS
LANE = 16
K = 128

_Z16 = None


def _zero_rows(ref, nrows, ncols):
  zv = jnp.zeros((LANE,), jnp.float32)

  def row(i, _):
    def col(c, __):
      ref[i, pl.ds(c * LANE, LANE)] = zv
      return 0
    lax.fori_loop(0, ncols // LANE, col, 0)
    return 0

  lax.fori_loop(0, nrows, row, 0)


def _deg_body(n_pad, rpt, n_chunks, dst_hbm, out_hbm, dst_v, ones_v, zbuf_v,
              deg_sh, sem):
  c = lax.axis_index("c")
  s = lax.axis_index("s")
  wid = c * NS + s

  pltpu.sync_copy(dst_hbm.at[pl.ds(wid * n_chunks, n_chunks)], dst_v)

  ov = jnp.ones((LANE,), jnp.float32)
  zv = jnp.zeros((LANE,), jnp.float32)

  def fill(i, _):
    ones_v[pl.ds(i * LANE, LANE)] = ov
    return 0
  lax.fori_loop(0, K // LANE, fill, 0)

  def zfill(i, _):
    zbuf_v[pl.ds(i * LANE, LANE)] = zv
    return 0
  lax.fori_loop(0, -(-rpt // LANE), zfill, 0)

  pltpu.sync_copy(zbuf_v.at[pl.ds(0, rpt)], deg_sh.at[pl.ds(s * rpt, rpt)])
  plsc.subcore_barrier()

  def chunk(j, _):
    pltpu.sync_copy(ones_v, deg_sh.at[dst_v.at[j]], add=True)
    return 0
  lax.fori_loop(0, n_chunks, chunk, 0)

  plsc.subcore_barrier()

  pltpu.sync_copy(deg_sh.at[pl.ds(s * rpt, rpt)],
                  out_hbm.at[pl.ds(c * n_pad + s * rpt, rpt)])


def _row_chunks(rpt):
  out = []
  off = 0
  while off < rpt:
    out.append((off, min(K, rpt - off)))
    off += K
  return out


def _agg_body(n_pad, d, rpt, n_chunks, g_hbm, src_hbm, dst_hbm, out_hbm,
              src_v, dst_v, rows_v, acc_sh, *sems):
  c = lax.axis_index("c")
  s = lax.axis_index("s")
  base = (c * NS + s) * n_chunks
  isem, dsem, gsem, ssem = sems[0:3], sems[3:6], sems[6:9], sems[9:12]

  def load_src(j, b):
    pltpu.async_copy(src_hbm.at[base + j], src_v.at[b], isem[b])

  def wait_src(b):
    pltpu.make_async_copy(src_hbm.at[0], src_v.at[b], isem[b]).wait()

  def load_dst(j, b):
    pltpu.async_copy(dst_hbm.at[base + j], dst_v.at[b], dsem[b])

  def wait_dst(b):
    pltpu.make_async_copy(dst_hbm.at[0], dst_v.at[b], dsem[b]).wait()

  def gather(j, b):
    pltpu.async_copy(g_hbm.at[src_v.at[b]], rows_v.at[b], gsem[b])

  def wait_gather(b):
    pltpu.make_async_copy(g_hbm.at[src_v.at[b]], rows_v.at[b],
                          gsem[b]).wait()

  def scatter(j, b):
    pltpu.async_copy(rows_v.at[b], acc_sh.at[dst_v.at[b]], ssem[b],
                     add=True)

  def wait_scatter(b):
    pltpu.make_async_copy(rows_v.at[b], acc_sh.at[pl.ds(0, K)],
                          ssem[b]).wait()

  zv = jnp.zeros((LANE,), jnp.float32)

  def zrow(i, _):
    def zcol(cc, __):
      rows_v[0, i, pl.ds(cc * LANE, LANE)] = zv
      return 0
    lax.fori_loop(0, d // LANE, zcol, 0)
    return 0
  lax.fori_loop(0, K, zrow, 0)
  for off, sz in _row_chunks(rpt):
    pltpu.sync_copy(rows_v.at[0, pl.ds(0, sz)],
                    acc_sh.at[pl.ds(s * rpt + off, sz)])
  plsc.subcore_barrier()

  def body(j, b):
    bp2 = (b + 2) % 3
    wait_gather(b)
    wait_dst(b)
    scatter(j, b)
    wait_src(bp2)
    wait_scatter(bp2)
    gather(j + 2, bp2)
    load_src(j + 3, b)
    load_dst(j + 2, bp2)

  nc = n_chunks
  load_src(0, 0)
  load_src(1, 1)
  load_src(2, 2)
  load_dst(0, 0)
  load_dst(1, 1)
  wait_src(0)
  gather(0, 0)
  wait_src(1)
  gather(1, 1)
  wait_gather(0)
  wait_dst(0)
  scatter(0, 0)
  wait_src(2)
  gather(2, 2)
  load_src(3, 0)
  load_dst(2, 2)

  nb = (nc - 3) // 3

  def block(i, _):
    for r in (1, 2, 3):
      body(3 * i + r, r % 3)
    return 0
  lax.fori_loop(0, nb, block, 0)
  for j in range(3 * nb + 1, nc - 2):
    body(j, j % 3)

  b = (nc - 2) % 3
  wait_gather(b)
  wait_dst(b)
  scatter(nc - 2, b)
  wait_scatter((nc - 3) % 3)
  b = (nc - 1) % 3
  wait_gather(b)
  wait_dst(b)
  scatter(nc - 1, b)
  wait_scatter((nc - 2) % 3)
  wait_scatter((nc - 1) % 3)
  wait_src(nc % 3)

  plsc.subcore_barrier()

  for off, sz in _row_chunks(rpt):
    pltpu.sync_copy(acc_sh.at[pl.ds(s * rpt + off, sz)],
                    out_hbm.at[pl.ds(c * n_pad + s * rpt + off, sz)])


@functools.lru_cache(maxsize=None)
def _make_deg(n_pad, n_chunks):
  rpt = n_pad // NS
  mesh = plsc.VectorSubcoreMesh(core_axis_name="c", subcore_axis_name="s",
                                num_cores=NC, num_subcores=NS)
  return pl.kernel(
      functools.partial(_deg_body, n_pad, rpt, n_chunks),
      out_type=jax.ShapeDtypeStruct((NC * n_pad,), jnp.float32),
      mesh=mesh,
      scratch_types=[
          pltpu.VMEM((n_chunks, K), jnp.int32),
          pltpu.VMEM((K,), jnp.float32),
          pltpu.VMEM((-(-rpt // LANE) * LANE,), jnp.float32),
          pltpu.VMEM_SHARED((n_pad,), jnp.float32),
          pltpu.SemaphoreType.DMA,
      ],
  )


@functools.lru_cache(maxsize=None)
def _make_agg(n_pad, d, n_chunks):
  rpt = n_pad // NS
  mesh = plsc.VectorSubcoreMesh(core_axis_name="c", subcore_axis_name="s",
                                num_cores=NC, num_subcores=NS)
  return pl.kernel(
      functools.partial(_agg_body, n_pad, d, rpt, n_chunks),
      out_type=jax.ShapeDtypeStruct((NC * n_pad, d), jnp.float32),
      mesh=mesh,
      scratch_types=[
          pltpu.VMEM((3, K), jnp.int32),
          pltpu.VMEM((3, K), jnp.int32),
          pltpu.VMEM((3, K, d), jnp.float32),
          pltpu.VMEM_SHARED((n_pad, d), jnp.float32),
      ] + [pltpu.SemaphoreType.DMA] * 12,
  )


def _tc1a_body(x_ref, w_ref, h_ref):
  h_ref[...] = jnp.dot(x_ref[...], w_ref[...],
                       preferred_element_type=jnp.float32)


def _tc1b_body(h_ref, d0_ref, d1_ref, g_ref):
  dinv = lax.rsqrt(d0_ref[...] + d1_ref[...] + 1.0)
  g_ref[...] = h_ref[...] * dinv


def _tc2_body(p0_ref, p1_ref, g_ref, b_ref, w_ref, d0_ref, d1_ref, o_ref):
  dinv = lax.rsqrt(d0_ref[...] + d1_ref[...] + 1.0)
  z = jnp.maximum(
      dinv * (p0_ref[0] + p1_ref[0] + g_ref[...]) + b_ref[...], 0.0)
  o_ref[...] = jnp.dot(z, w_ref[...], preferred_element_type=jnp.float32) * dinv


def _tc3_body(p0_ref, p1_ref, g_ref, b_ref, d0_ref, d1_ref, o_ref):
  dinv = lax.rsqrt(d0_ref[...] + d1_ref[...] + 1.0)
  o_ref[...] = jnp.maximum(
      dinv * (p0_ref[0] + p1_ref[0] + g_ref[...]) + b_ref[...], 0.0)


def _row_spec(bn, d):
  return pl.BlockSpec((bn, d), lambda i: (i, 0))


def _full_spec(shape):
  return pl.BlockSpec(shape, lambda i: tuple(0 for _ in shape))


def kernel(x, edge_index, W1, b1, W2, b2):
  n, d_in = x.shape
  hid = W1.shape[1]
  e = edge_index.shape[1]

  src = edge_index[0].astype(jnp.int32)
  dst = edge_index[1].astype(jnp.int32)

  n_pad = -(-(n + 1) // (NS * 8)) * (NS * 8)
  n_pad_deg = -(-(n + 1) // (NS * 16)) * (NS * 16)

  e_pad = -(-e // (NW * K * 8)) * (NW * K * 8)
  pad = e_pad - e
  if pad:
    spread = jnp.arange(pad, dtype=jnp.int32)
    src = jnp.concatenate([src, spread % n])
    dst = jnp.concatenate([dst, n + spread % (n_pad - n)])
  n_chunks = e_pad // (NW * K)
  src2d = jnp.concatenate(
      [src.reshape(NW * n_chunks, K), jnp.zeros((8, K), jnp.int32)])
  dst2d = dst.reshape(NW * n_chunks, K)

  deg = _make_deg(n_pad_deg, n_chunks)(dst2d).reshape(NC, n_pad_deg)
  d0 = deg[0, :n][:, None]
  d1 = deg[1, :n][:, None]

  bn = 2000 if n % 2000 == 0 else n
  grid = (n // bn,)
  b1r = b1.reshape(1, hid)
  b2r = b2.reshape(1, hid)

  h1 = pl.pallas_call(
      _tc1a_body,
      grid=grid,
      in_specs=[_row_spec(bn, d_in), _full_spec((d_in, hid))],
      out_specs=_row_spec(bn, hid),
      out_shape=jax.ShapeDtypeStruct((n, hid), jnp.float32),
  )(x, W1)

  g1 = pl.pallas_call(
      _tc1b_body,
      grid=grid,
      in_specs=[_row_spec(bn, hid), _row_spec(bn, 1), _row_spec(bn, 1)],
      out_specs=_row_spec(bn, hid),
      out_shape=jax.ShapeDtypeStruct((n, hid), jnp.float32),
  )(h1, d0, d1)

  part0 = pl.BlockSpec((1, bn, hid), lambda i: (0, i, 0))
  part1 = pl.BlockSpec((1, bn, hid), lambda i: (1, i, 0))

  agg = _make_agg(n_pad, hid, n_chunks)
  p = agg(g1, src2d, dst2d).reshape(NC, n_pad, hid)

  g2 = pl.pallas_call(
      _tc2_body,
      grid=grid,
      in_specs=[part0, part1, _row_spec(bn, hid),
                _full_spec((1, hid)), _full_spec((hid, hid)),
                _row_spec(bn, 1), _row_spec(bn, 1)],
      out_specs=_row_spec(bn, hid),
      out_shape=jax.ShapeDtypeStruct((n, hid), jnp.float32),
  )(p, p, g1, b1r, W2, d0, d1)

  q = agg(g2, src2d, dst2d).reshape(NC, n_pad, hid)

  out = pl.pallas_call(
      _tc3_body,
      grid=grid,
      in_specs=[part0, part1, _row_spec(bn, hid),
                _full_spec((1, hid)), _row_spec(bn, 1), _row_spec(bn, 1)],
      out_specs=_row_spec(bn, hid),
      out_shape=jax.ShapeDtypeStruct((n, hid), jnp.float32),
  )(q, q, g2, b2r, d0, d1)

  return out
